# baseline (speedup 1.0000x reference)
import jax
import jax.numpy as jnp
from jax import lax
from jax.experimental import pallas as pl
from jax.experimental.pallas import tpu as pltpu
from jax.experimental.pallas import tpu_sc as plsc

_N = 10000
_E = 320000
_ETOT = _E + _N
_DIN = 128
_D = 64
_L = 10
_NC = 2
_NS = 16
_CHUNK = 128
_CHUNKS = 162
_EPW = _CHUNKS * _CHUNK
_EPAD = _NS * _EPW
_NW = 4
_WROWS = 2560
_NPAD = _NW * _WROWS
_WRPT = _WROWS // _NS
_TROWS = _NPAD // _NS
_SCAP = 1536
_SCAPV = _SCAP // 16
_LCAPW = 6144
_LCAPWC = _LCAPW // _CHUNK
_LTOT = _NW * _LCAPW


def _sc_scan_body(src_hbm, dst_hbm, seg_src_hbm, seg_dst_hbm, seg_cnt_hbm,
                  src_v, dst_v, ssrc_v, sdst_v, cbuf_v):
    c = lax.axis_index("c")
    s = lax.axis_index("s")
    wid = c * _NS + s
    pltpu.sync_copy(src_hbm.at[pl.ds(wid * _EPW, _EPW)], src_v)
    pltpu.sync_copy(dst_hbm.at[pl.ds(wid * _EPW, _EPW)], dst_v)

    def scan_body(i, cnts):
        srcv = src_v[pl.ds(i * 16, 16)]
        dstv = dst_v[pl.ds(i * 16, 16)]
        new = []
        for t in range(_NS):
            lo = t * _TROWS
            m = (srcv >= lo) & (srcv < lo + _TROWS)
            csum = plsc.cumsum(m.astype(jnp.int32))
            pos = t * _SCAP + cnts[t] + csum - 1
            plsc.store_scatter(ssrc_v, [pos], srcv - lo, mask=m)
            plsc.store_scatter(sdst_v, [pos], dstv, mask=m)
            pc = plsc.all_reduce_population_count(m)
            new.append(jnp.minimum(cnts[t] + pc, _SCAP - 16))
        return tuple(new)

    zero16 = jnp.zeros((16,), jnp.int32)
    cnts = lax.fori_loop(0, _EPW // 16, scan_body, (zero16,) * _NS)
    for t in range(_NS):
        cbuf_v[pl.ds(t * 16, 16)] = cnts[t]
    pltpu.sync_copy(ssrc_v, seg_src_hbm.at[pl.ds(wid * _NS * _SCAP, _NS * _SCAP)])
    pltpu.sync_copy(sdst_v, seg_dst_hbm.at[pl.ds(wid * _NS * _SCAP, _NS * _SCAP)])
    pltpu.sync_copy(cbuf_v, seg_cnt_hbm.at[pl.ds(wid * _NS * 16, _NS * 16)])


def _sc_consol_body(seg_src_hbm, seg_dst_hbm, seg_cnt_hbm, fsrc_hbm, fdst_hbm,
                    srcl_hbm, dstl_hbm,
                    segs_v, segd_v, cbuf_v, srcl_v, dstl_v):
    c = lax.axis_index("c")
    t = lax.axis_index("s")
    wid = c * _NS + t
    pltpu.sync_copy(fsrc_hbm, srcl_v)
    pltpu.sync_copy(fdst_hbm, dstl_v)
    lane = lax.iota(jnp.int32, 16)

    def seg_body(sl, wcnts):
        base = (c * _NS + sl) * _NS * _SCAP + t * _SCAP
        pltpu.sync_copy(seg_src_hbm.at[pl.ds(base, _SCAP)], segs_v)
        pltpu.sync_copy(seg_dst_hbm.at[pl.ds(base, _SCAP)], segd_v)
        pltpu.sync_copy(
            seg_cnt_hbm.at[pl.ds((c * _NS + sl) * _NS * 16 + t * 16, 16)], cbuf_v)
        cntspl = cbuf_v[pl.ds(0, 16)]

        def vreg_body(r, wc):
            posv = r * 16 + lane
            valid = posv < cntspl
            srcv = segs_v[pl.ds(r * 16, 16)]
            dstv = segd_v[pl.ds(r * 16, 16)]
            new = []
            for w in range(_NW):
                lo = w * _WROWS
                m = valid & (dstv >= lo) & (dstv < lo + _WROWS)
                csum = plsc.cumsum(m.astype(jnp.int32))
                pos = w * _LCAPW + wc[w] + csum - 1
                plsc.store_scatter(srcl_v, [pos], srcv, mask=m)
                plsc.store_scatter(dstl_v, [pos], dstv - lo, mask=m)
                pc = plsc.all_reduce_population_count(m)
                new.append(jnp.minimum(wc[w] + pc, _LCAPW - 16))
            return tuple(new)

        return lax.fori_loop(0, _SCAPV, vreg_body, wcnts)

    zero16 = jnp.zeros((16,), jnp.int32)
    lax.fori_loop(0, _NS, seg_body, (zero16,) * _NW)
    pltpu.sync_copy(srcl_v, srcl_hbm.at[pl.ds(wid * _LTOT, _LTOT)])
    pltpu.sync_copy(dstl_v, dstl_hbm.at[pl.ds(wid * _LTOT, _LTOT)])


def _sc_spmm_body(g_hbm, srcl_hbm, dstl_hbm, z_hbm, out_hbm,
                  srcl_v, dstl_v, gsl_v, mbuf_v, acc_sh, ssem):
    c = lax.axis_index("c")
    s = lax.axis_index("s")
    wid = c * _NS + s
    r0 = s * _WRPT
    pltpu.sync_copy(srcl_hbm.at[pl.ds(wid * _LTOT, _LTOT)], srcl_v)
    pltpu.sync_copy(dstl_hbm.at[c, s], dstl_v)
    g2 = g_hbm.at[c]
    pltpu.sync_copy(g2.at[pl.ds(s * _TROWS, _TROWS)], gsl_v.at[pl.ds(0, _TROWS)])
    zv = jnp.zeros((16,), jnp.float32)
    for k in range(_D // 16):
        gsl_v[_TROWS, pl.ds(k * 16, 16)] = zv
    pltpu.sync_copy(z_hbm, acc_sh.at[pl.ds(r0, _WRPT)])
    plsc.subcore_barrier()
    lane = lax.iota(jnp.int32, 16)
    adum = acc_sh.at[pl.ds(0, _CHUNK)]
    for w in range(_NW):

        def pair_body(j2, carry):
            for b in range(2):
                j = 2 * j2 + b

                @pl.when(j2 > 0)
                def _():
                    pltpu.make_async_copy(mbuf_v.at[b], adum, ssem.at[b]).wait()

                base = w * _LCAPW + j * _CHUNK

                def sub_body(sub, inner):
                    srcv = srcl_v[pl.ds(base + sub * 16, 16)]
                    rowpos = sub * 16 + lane
                    for f in range(_D):
                        fspl = jnp.full((16,), f, jnp.int32)
                        val = plsc.load_gather(gsl_v, [srcv, fspl])
                        plsc.store_scatter(mbuf_v.at[b], [rowpos, fspl], val)
                    return inner

                lax.fori_loop(0, _CHUNK // 16, sub_body, 0)
                pltpu.async_copy(mbuf_v.at[b], acc_sh.at[dstl_v.at[w, j]],
                                 ssem.at[b], add=True)
            return carry

        lax.fori_loop(0, _LCAPWC // 2, pair_body, 0)
        for b in range(2):
            pltpu.make_async_copy(mbuf_v.at[b], adum, ssem.at[b]).wait()
        plsc.subcore_barrier()
        pltpu.sync_copy(acc_sh.at[pl.ds(r0, _WRPT)],
                        out_hbm.at[c].at[pl.ds(w * _WROWS + r0, _WRPT)])
        if w + 1 < _NW:
            pltpu.sync_copy(z_hbm, acc_sh.at[pl.ds(r0, _WRPT)])
            plsc.subcore_barrier()


_sc_calls_cache = {}


def _sc_calls():
    if "scan" not in _sc_calls_cache:
        mesh = plsc.VectorSubcoreMesh(core_axis_name="c", subcore_axis_name="s",
                                      num_cores=_NC, num_subcores=_NS)
        params = pltpu.CompilerParams(use_tc_tiling_on_sc=False,
                                      needs_layout_passes=False)
        _sc_calls_cache["scan"] = pl.kernel(
            _sc_scan_body,
            out_type=(
                jax.ShapeDtypeStruct((_NC * _NS * _NS * _SCAP,), jnp.int32),
                jax.ShapeDtypeStruct((_NC * _NS * _NS * _SCAP,), jnp.int32),
                jax.ShapeDtypeStruct((_NC * _NS * _NS * 16,), jnp.int32),
            ),
            mesh=mesh,
            compiler_params=params,
            scratch_types=[
                pltpu.VMEM((_EPW,), jnp.int32),
                pltpu.VMEM((_EPW,), jnp.int32),
                pltpu.VMEM((_NS * _SCAP,), jnp.int32),
                pltpu.VMEM((_NS * _SCAP,), jnp.int32),
                pltpu.VMEM((_NS * 16,), jnp.int32),
            ],
        )
        _sc_calls_cache["consol"] = pl.kernel(
            _sc_consol_body,
            out_type=(
                jax.ShapeDtypeStruct((_NC * _NS * _LTOT,), jnp.int32),
                jax.ShapeDtypeStruct((_NC * _NS * _LTOT,), jnp.int32),
            ),
            mesh=mesh,
            compiler_params=params,
            scratch_types=[
                pltpu.VMEM((_SCAP,), jnp.int32),
                pltpu.VMEM((_SCAP,), jnp.int32),
                pltpu.VMEM((16,), jnp.int32),
                pltpu.VMEM((_LTOT,), jnp.int32),
                pltpu.VMEM((_LTOT,), jnp.int32),
            ],
        )
        _sc_calls_cache["spmm"] = pl.kernel(
            _sc_spmm_body,
            out_type=jax.ShapeDtypeStruct((_NC, _NPAD, _D), jnp.float32),
            mesh=mesh,
            compiler_params=params,
            scratch_types=[
                pltpu.VMEM((_LTOT,), jnp.int32),
                pltpu.VMEM((_NW, _LCAPWC, _CHUNK), jnp.int32),
                pltpu.VMEM((_TROWS + 8, _D), jnp.float32),
                pltpu.VMEM((2, _CHUNK, _D), jnp.float32),
                pltpu.VMEM_SHARED((_WROWS, _D), jnp.float32),
                pltpu.SemaphoreType.DMA((2,)),
            ],
        )
    return _sc_calls_cache


def _rowmask():
    return lax.broadcasted_iota(jnp.int32, (_NPAD, 1), 0) < _N


def _dis(deg_ref, c):
    return lax.rsqrt(jnp.maximum(deg_ref[c, :, 0:1], 1.0))


def _tc_prep_body(xp_ref, w0_ref, deg_ref, g_ref):
    mask = _rowmask()
    for c in range(_NC):
        xw = jnp.dot(xp_ref[...], w0_ref[c], preferred_element_type=jnp.float32)
        g_ref[c] = jnp.where(mask, _dis(deg_ref, c) * xw, 0.0)


def _tc_step_body(s_ref, deg_ref, b_ref, w_ref, g_ref):
    mask = _rowmask()
    for c in range(_NC):
        dis = _dis(deg_ref, c)
        h = jnp.maximum(dis * s_ref[c] + b_ref[c], 0.0)
        g_ref[c] = jnp.where(
            mask, dis * jnp.dot(h, w_ref[c], preferred_element_type=jnp.float32), 0.0)


def _tc_final_body(s_ref, deg_ref, b_ref, watt_ref, out_ref):
    feats = []
    for c in range(_NC):
        feats.append(_dis(deg_ref, c) * s_ref[c] + b_ref[c])
    nf = jnp.concatenate(feats, axis=1)
    nrm = lax.rsqrt(jnp.sum(nf * nf, axis=1, keepdims=True))
    nfn = nf * nrm
    mask = _rowmask()
    nfn_m = jnp.where(mask, nfn, 0.0)
    mean = jnp.sum(nfn_m, axis=0, keepdims=True) * (1.0 / _N)
    ctx = jnp.tanh(jnp.dot(mean, watt_ref[...], preferred_element_type=jnp.float32))
    score = jax.nn.sigmoid(jnp.sum(nfn_m * ctx, axis=1, keepdims=True))
    gf = jnp.sum(jnp.where(mask, score * nfn_m, 0.0), axis=0, keepdims=True)
    out_ref[0] = jnp.concatenate(
        [nfn_m, jnp.broadcast_to(gf, (_NPAD, 2 * _D))], axis=1)


def _prep_call(xp, w0s, deg):
    return pl.pallas_call(
        _tc_prep_body,
        out_shape=jax.ShapeDtypeStruct((_NC, _NPAD, _D), jnp.float32),
    )(xp, w0s, deg)


def _step_call(sk, deg, bk, wk):
    return pl.pallas_call(
        _tc_step_body,
        out_shape=jax.ShapeDtypeStruct((_NC, _NPAD, _D), jnp.float32),
    )(sk, deg, bk, wk)


def _final_call(s9, deg, b9, watt):
    return pl.pallas_call(
        _tc_final_body,
        out_shape=jax.ShapeDtypeStruct((1, _NPAD, 4 * _D), jnp.float32),
    )(s9, deg, b9, watt)


def kernel(x, edge_index, batch, Wf0, bf0, Wf, bf, Wr0, br0, Wr, br, Watt):
    loopv = jnp.arange(_N, dtype=jnp.int32)
    padv = jnp.full((_EPAD - _ETOT,), jnp.int32(1 << 30), jnp.int32)
    a = jnp.concatenate([edge_index[0], loopv, padv])
    b = jnp.concatenate([edge_index[1], loopv, padv])
    src2 = jnp.stack([a, b]).reshape(-1)
    dst2 = jnp.stack([b, a]).reshape(-1)

    fsrc = jnp.full((_LTOT,), _TROWS, jnp.int32)
    fdst = jnp.zeros((_LTOT,), jnp.int32)
    zwin = jnp.zeros((_WRPT, _D), jnp.float32)
    rmask = (jnp.arange(_NPAD) < _N).astype(jnp.float32)[:, None]
    ones_g = jnp.broadcast_to(rmask, (_NPAD, _D))[None] * jnp.ones((_NC, 1, 1), jnp.float32)
    xp = jnp.pad(x, ((0, _NPAD - _N), (0, 0)))

    w0s = jnp.stack([Wf0, Wr0])
    wks = jnp.stack([Wf, Wr])
    b0 = jnp.stack([bf0, br0])
    bks = jnp.stack([bf, br])

    sc = _sc_calls()
    seg_src, seg_dst, seg_cnt = sc["scan"](src2, dst2)
    srcl, dstl = sc["consol"](seg_src, seg_dst, seg_cnt, fsrc, fdst)
    dstl = dstl.reshape(_NC, _NS, _NW, _LCAPWC, _CHUNK)

    deg = sc["spmm"](ones_g, srcl, dstl, zwin)
    g = _prep_call(xp, w0s, deg)
    for k in range(_L - 1):
        sk = sc["spmm"](g, srcl, dstl, zwin)
        bk = b0 if k == 0 else bks[:, k - 1]
        g = _step_call(sk, deg, bk, wks[:, k])
    s9 = sc["spmm"](g, srcl, dstl, zwin)
    out = _final_call(s9, deg, bks[:, _L - 2], Watt)
    return out[:, :_N, :]
